# baseline (speedup 1.0000x reference)
import jax
import jax.numpy as jnp
from jax import lax
from jax.experimental import pallas as pl
from jax.experimental.pallas import tpu as pltpu
from jax.experimental.pallas import tpu_sc as plsc

_DIM = 32
_BATCH = 16384
_HIST = 50
_IW = 128
_NC = 2
_NS = 16
_NW = _NC * _NS
_CPW = (_BATCH // _IW) // _NW
_D4 = _DIM // 8


def _body(idx_hbm, tbl_hbm, out_hbm, idx_v, g_v, t_v, sem_i, sem_g, sem_o):
    wid = lax.axis_index("s") * _NC + lax.axis_index("c")
    b0 = wid * (_CPW * _IW)
    c0 = wid * _CPW

    lane = lax.iota(jnp.int32, 16)
    dcol = [jnp.full((16,), d, jnp.int32) for d in range(_DIM)]

    def idx_copy(h, slot):
        return pltpu.make_async_copy(
            idx_hbm.at[h, pl.ds(b0, _CPW * _IW)], idx_v.at[slot], sem_i)

    def out_copy(h, slot, j):
        return pltpu.make_async_copy(
            t_v.at[slot].at[j], out_hbm.at[h, :, c0 + j], sem_o)

    def transpose_item(slot, j):
        gref = g_v.at[slot].at[j]
        tref = t_v.at[slot].at[j]

        @plsc.parallel_loop(0, _IW // 16, 1, unroll=2)
        def _tb(bg):
            bv = bg * 16 + lane
            for d in range(_DIM):
                v = plsc.load_gather(gref, [bv, dcol[d]])
                tref[d >> 3, d & 7, pl.ds(bg * 16, 16)] = v

    idx_copy(0, 0).start()
    idx_copy(1, 1).start()
    for slot in range(2):
        for j in range(_CPW):
            out_copy(slot, slot, j).start()

    def step(i, carry):
        for slot in range(2):
            h = 2 * i + slot
            idx_copy(h, slot).wait()
            gc = [
                pltpu.make_async_copy(
                    tbl_hbm.at[idx_v.at[slot].at[pl.ds(j * _IW, _IW)]],
                    g_v.at[slot].at[j], sem_g)
                for j in range(_CPW)
            ]
            for c in gc:
                c.start()
            for c in gc:
                c.wait()
            idx_copy(jnp.minimum(h + 2, _HIST - 1), slot).start()
            for j in range(_CPW):
                out_copy(h, slot, j).wait()
                transpose_item(slot, j)
                out_copy(h, slot, j).start()
        return carry

    lax.fori_loop(0, _HIST // 2, step, 0)

    for slot in range(2):
        idx_copy(_HIST - 1, slot).wait()
        for j in range(_CPW):
            out_copy(_HIST - 2 + slot, slot, j).wait()


def _embed(idx_t, weight):
    k = pl.kernel(
        _body,
        out_type=jax.ShapeDtypeStruct((_HIST, _D4, _BATCH // _IW, 8, _IW),
                                      jnp.float32),
        mesh=plsc.VectorSubcoreMesh(core_axis_name="c", subcore_axis_name="s"),
        scratch_types=[
            pltpu.VMEM((2, _CPW * _IW), jnp.int32),
            pltpu.VMEM((2, _CPW, _IW, _DIM), jnp.float32),
            pltpu.VMEM((2, _CPW, _D4, 8, _IW), jnp.float32),
            pltpu.SemaphoreType.DMA,
            pltpu.SemaphoreType.DMA,
            pltpu.SemaphoreType.DMA,
        ],
        compiler_params=pltpu.CompilerParams(use_tc_tiling_on_sc=False,
                                             needs_layout_passes=False),
    )
    return k(idx_t, weight)


def kernel(input, weight):
    idx_t = input.astype(jnp.int32).T
    out5 = _embed(idx_t, weight)
    out = out5.transpose(2, 4, 0, 1, 3).reshape(_BATCH, _HIST, _DIM)
    return out


# speedup vs baseline: 1.1013x; 1.1013x over previous
import jax
import jax.numpy as jnp
from jax import lax
from jax.experimental import pallas as pl
from jax.experimental.pallas import tpu as pltpu
from jax.experimental.pallas import tpu_sc as plsc

_DIM = 32
_BATCH = 16384
_HIST = 50
_IW = 128
_NC = 2
_NS = 16
_NW = _NC * _NS
_CPW = (_BATCH // _IW) // _NW
_D4 = _DIM // 8


def _body(idx_hbm, tbl_hbm, out_hbm, idx_v, g_v, t_v, sem_i, sem_g, sem_o):
    wid = lax.axis_index("s") * _NC + lax.axis_index("c")
    b0 = wid * (_CPW * _IW)
    c0 = wid * _CPW

    lane = lax.iota(jnp.int32, 16)
    d4v = [(lane >> 3) + 2 * dg for dg in range(2)]
    sv = lane & 7

    def idx_copy(h, slot):
        return pltpu.make_async_copy(
            idx_hbm.at[h, pl.ds(b0, _CPW * _IW)], idx_v.at[slot], sem_i)

    def out_copy(h, slot, j):
        return pltpu.make_async_copy(
            t_v.at[slot].at[j], out_hbm.at[h, :, c0 + j], sem_o)

    def transpose_item(slot, j):
        gref = g_v.at[slot].at[j]
        tref = t_v.at[slot].at[j]

        @plsc.parallel_loop(0, _IW, 1, unroll=16)
        def _tb(b):
            bv = jnp.full((16,), b, jnp.int32)
            for dg in range(2):
                v = gref[b, pl.ds(dg * 16, 16)]
                plsc.store_scatter(tref, [d4v[dg], sv, bv], v)

    idx_copy(0, 0).start()
    idx_copy(1, 1).start()
    for slot in range(2):
        for j in range(_CPW):
            out_copy(slot, slot, j).start()

    def step(i, carry):
        gcs = []
        for slot in range(2):
            h = 2 * i + slot
            idx_copy(h, slot).wait()
            gc = [
                pltpu.make_async_copy(
                    tbl_hbm.at[idx_v.at[slot].at[pl.ds(j * _IW, _IW)]],
                    g_v.at[slot].at[j], sem_g)
                for j in range(_CPW)
            ]
            for c in gc:
                c.start()
            gcs.append(gc)
        for slot in range(2):
            h = 2 * i + slot
            for c in gcs[slot]:
                c.wait()
            idx_copy(jnp.minimum(h + 2, _HIST - 1), slot).start()
            for j in range(_CPW):
                out_copy(h, slot, j).wait()
                transpose_item(slot, j)
                out_copy(h, slot, j).start()
        return carry

    lax.fori_loop(0, _HIST // 2, step, 0)

    for slot in range(2):
        idx_copy(_HIST - 1, slot).wait()
        for j in range(_CPW):
            out_copy(_HIST - 2 + slot, slot, j).wait()


def _embed(idx_t, weight):
    k = pl.kernel(
        _body,
        out_type=jax.ShapeDtypeStruct((_HIST, _D4, _BATCH // _IW, 8, _IW),
                                      jnp.float32),
        mesh=plsc.VectorSubcoreMesh(core_axis_name="c", subcore_axis_name="s"),
        scratch_types=[
            pltpu.VMEM((2, _CPW * _IW), jnp.int32),
            pltpu.VMEM((2, _CPW, _IW, _DIM), jnp.float32),
            pltpu.VMEM((2, _CPW, _D4, 8, _IW), jnp.float32),
            pltpu.SemaphoreType.DMA,
            pltpu.SemaphoreType.DMA,
            pltpu.SemaphoreType.DMA,
        ],
        compiler_params=pltpu.CompilerParams(use_tc_tiling_on_sc=False,
                                             needs_layout_passes=False),
    )
    return k(idx_t, weight)


def kernel(input, weight):
    idx_t = input.astype(jnp.int32).T
    out5 = _embed(idx_t, weight)
    out = out5.transpose(2, 4, 0, 1, 3).reshape(_BATCH, _HIST, _DIM)
    return out


# speedup vs baseline: 1.6250x; 1.4756x over previous
import jax
import jax.numpy as jnp
from jax import lax
from jax.experimental import pallas as pl
from jax.experimental.pallas import tpu as pltpu
from jax.experimental.pallas import tpu_sc as plsc

_DIM = 32
_BATCH = 16384
_HIST = 50
_IW = 128
_NC = 2
_NS = 16
_NW = _NC * _NS
_CPW = (_BATCH // _IW) // _NW
_D4 = _DIM // 8


def _body(idx_hbm, tbl_hbm, out_hbm, idx_v, g_v, t_v, sem_i, sem_g, sem_o):
    wid = lax.axis_index("s") * _NC + lax.axis_index("c")
    b0 = wid * (_CPW * _IW)
    c0 = wid * _CPW

    lane = lax.iota(jnp.int32, 16)
    d4v = [(lane >> 3) + 2 * dg for dg in range(2)]
    sv = lane & 7

    def idx_copy(h, slot):
        return pltpu.make_async_copy(
            idx_hbm.at[h, pl.ds(b0, _CPW * _IW)], idx_v.at[slot], sem_i)

    def out_copy(h, slot, j):
        return pltpu.make_async_copy(
            t_v.at[slot, j, :, :, pl.ds(0, _IW)],
            out_hbm.at[h, :, c0 + j], sem_o)

    def transpose_item(slot, j):
        gref = g_v.at[slot].at[j]
        tref = t_v.at[slot].at[j]

        @plsc.parallel_loop(0, _IW, 1, unroll=16)
        def _tb(b):
            bv = jnp.full((16,), b, jnp.int32)
            for dg in range(2):
                v = gref[b, pl.ds(dg * 16, 16)]
                plsc.store_scatter(tref, [d4v[dg], sv, bv], v)

    idx_copy(0, 0).start()
    idx_copy(1, 1).start()
    for slot in range(2):
        for j in range(_CPW):
            out_copy(slot, slot, j).start()

    def step(i, carry):
        gcs = []
        for slot in range(2):
            h = 2 * i + slot
            idx_copy(h, slot).wait()
            gc = [
                pltpu.make_async_copy(
                    tbl_hbm.at[idx_v.at[slot].at[pl.ds(j * _IW, _IW)]],
                    g_v.at[slot].at[j], sem_g)
                for j in range(_CPW)
            ]
            for c in gc:
                c.start()
            gcs.append(gc)
        for slot in range(2):
            h = 2 * i + slot
            for c in gcs[slot]:
                c.wait()
            idx_copy(jnp.minimum(h + 2, _HIST - 1), slot).start()
            for j in range(_CPW):
                out_copy(h, slot, j).wait()
                transpose_item(slot, j)
                out_copy(h, slot, j).start()
        return carry

    lax.fori_loop(0, _HIST // 2, step, 0)

    for slot in range(2):
        idx_copy(_HIST - 1, slot).wait()
        for j in range(_CPW):
            out_copy(_HIST - 2 + slot, slot, j).wait()


def _embed(idx_t, weight):
    k = pl.kernel(
        _body,
        out_type=jax.ShapeDtypeStruct((_HIST, _D4, _BATCH // _IW, 8, _IW),
                                      jnp.float32),
        mesh=plsc.VectorSubcoreMesh(core_axis_name="c", subcore_axis_name="s"),
        scratch_types=[
            pltpu.VMEM((2, _CPW * _IW), jnp.int32),
            pltpu.VMEM((2, _CPW, _IW, _DIM), jnp.float32),
            pltpu.VMEM((2, _CPW, _D4, 8, _IW + 1), jnp.float32),
            pltpu.SemaphoreType.DMA,
            pltpu.SemaphoreType.DMA,
            pltpu.SemaphoreType.DMA,
        ],
        compiler_params=pltpu.CompilerParams(use_tc_tiling_on_sc=False,
                                             needs_layout_passes=False),
    )
    return k(idx_t, weight)


def kernel(input, weight):
    idx_t = input.astype(jnp.int32).T
    out5 = _embed(idx_t, weight)
    out = out5.transpose(2, 4, 0, 1, 3).reshape(_BATCH, _HIST, _DIM)
    return out
